# baseline (speedup 1.0000x reference)
import functools

import jax
import jax.numpy as jnp
from jax import lax
from jax.experimental import pallas as pl
from jax.experimental.pallas import tpu as pltpu
from jax.experimental.pallas import tpu_sc as plsc

I_T = 10000
J_W = 1000
K_C = 32
N_L = 10000

NC = 2
NS = 16

HALF = I_T // NC
IC_PAD = 5120
LBL_OWN = N_L // NS
LBL_W = 640
CH = 64
NCH = LBL_W // CH
ROWS = 320
ZW = K_C * IC_PAD // NS
LN2 = 0.6931471805599453


def _log_f32(x):
    bits = plsc.bitcast(x, jnp.int32)
    e = (bits >> 23) - 127
    m = plsc.bitcast((bits & 0x7FFFFF) | 0x3F800000, jnp.float32)
    s = (m - 1.0) / (m + 1.0)
    s2 = s * s
    p = 1.0 + s2 * (0.3333333333 + s2 * (0.2 + s2 * (0.14285714 + s2 * 0.11111111)))
    return e.astype(jnp.float32) * LN2 + 2.0 * s * p


def _vq_body(ii_hbm, jj_hbm, y_hbm, snr_hbm, zer_hbm,
             qz_out, vq_out,
             ii_v, jj_v, y_v, a_v, d_v, snr_v, idx_b, val_b, iib_b, av_b,
             colb, qzb, bb, vqb,
             scat_sh, base_sh, a_sh, d_sh, sem, ssem):
    c = lax.axis_index("c")
    s = lax.axis_index("s")
    own_lo = s * LBL_OWN
    l0 = jnp.minimum(own_lo & ~7, N_L - LBL_W)
    l0 = pl.multiple_of(l0, 8)
    c_lo = c * HALF

    w0 = jnp.minimum(s * 64, J_W - 64)
    w0 = pl.multiple_of(w0, 8)
    pltpu.sync_copy(snr_hbm.at[pl.ds(w0, 64)], snr_v)

    cps = [
        pltpu.async_copy(ii_hbm.at[pl.ds(l0, LBL_W)], ii_v, sem),
        pltpu.async_copy(jj_hbm.at[pl.ds(l0, LBL_W)], jj_v, sem),
        pltpu.async_copy(y_hbm.at[pl.ds(l0, LBL_W)], y_v, sem),
        pltpu.async_copy(zer_hbm, scat_sh.at[pl.ds(s * ZW, ZW)], sem),
        pltpu.async_copy(zer_hbm.at[pl.ds(0, 640)],
                         base_sh.at[pl.ds((s % 8) * 640, 640)], sem),
    ]
    for v in range(4):
        x = snr_v[pl.ds(v * 16, 16)]
        sig = 1.0 / (1.0 + jnp.exp(-x))
        noi = 1.0 / (1.0 + jnp.exp(x))
        tot = sig + noi
        a = _log_f32(noi / (K_C * tot))
        b = _log_f32((sig + noi / K_C) / tot)
        a_v[pl.ds(v * 16, 16)] = a
        d_v[pl.ds(v * 16, 16)] = b - a
    pltpu.sync_copy(a_v.at[pl.ds(0, 64)], a_sh.at[pl.ds(w0, 64)])
    pltpu.sync_copy(d_v.at[pl.ds(0, 64)], d_sh.at[pl.ds(w0, 64)])
    for cp in cps:
        cp.wait()
    plsc.subcore_barrier()
    cps = [
        pltpu.async_copy(a_sh.at[pl.ds(0, J_W)], a_v, sem),
        pltpu.async_copy(d_sh.at[pl.ds(0, J_W)], d_v, sem),
    ]
    for cp in cps:
        cp.wait()

    zero = jnp.zeros((16,), jnp.float32)
    scatter_cps = []
    for chunk in range(NCH):
        for v in range(CH // 16):
            off = chunk * CH + v * 16
            iiv = ii_v[pl.ds(off, 16)]
            jjv = jj_v[pl.ds(off, 16)]
            yv = y_v[pl.ds(off, 16)]
            av = plsc.load_gather(a_v, [jjv])
            dv = plsc.load_gather(d_v, [jjv])
            g = l0 + off + lax.iota(jnp.int32, 16)
            col = iiv - c_lo
            ok = (g >= own_lo) & (g < own_lo + LBL_OWN) \
                & (col >= 0) & (col < HALF)
            colc = jnp.where(ok, col, 0)
            neg1 = jnp.full((16,), -1, jnp.int32)
            idx_b[chunk, pl.ds(v * 16, 16)] = jnp.where(
                ok, yv * IC_PAD + colc, neg1)
            val_b[chunk, pl.ds(v * 16, 16)] = jnp.where(ok, dv, zero)
            iib_b[chunk, pl.ds(v * 16, 16)] = jnp.where(ok, colc, neg1)
            av_b[chunk, pl.ds(v * 16, 16)] = jnp.where(ok, av, zero)
        scatter_cps.append(
            pltpu.async_copy(
                val_b.at[chunk],
                scat_sh.at[plsc.Indices(idx_b.at[chunk], ignored_value=-1)],
                ssem, add=True))
        scatter_cps.append(
            pltpu.async_copy(
                av_b.at[chunk],
                base_sh.at[plsc.Indices(iib_b.at[chunk], ignored_value=-1)],
                ssem, add=True))
    for cp in scatter_cps:
        cp.wait()
    plsc.subcore_barrier()

    r0l = jnp.minimum(s * 313, HALF - ROWS) & ~7
    r0l = pl.multiple_of(r0l, 8)
    col_cps = [
        pltpu.async_copy(scat_sh.at[pl.ds(k * IC_PAD + r0l, ROWS)],
                         colb.at[pl.ds(k * ROWS, ROWS)], sem)
        for k in range(K_C)
    ]
    col_cps.append(pltpu.async_copy(base_sh.at[pl.ds(r0l, ROWS)], bb, sem))
    for cp in col_cps:
        cp.wait()

    def _tree(xs, op):
        while len(xs) > 1:
            xs = [op(xs[i], xs[i + 1]) for i in range(0, len(xs) - 1, 2)] + (
                [xs[-1]] if len(xs) % 2 else [])
        return xs[0]

    def group(g, carry):
        vs = [colb[pl.ds(k * ROWS + g * 16, 16)] for k in range(K_C)]
        m = _tree(vs, jnp.maximum)
        es = [jnp.exp(v - m) for v in vs]
        z = _tree(es, lambda a, b: a + b)
        r = 1.0 / z
        rows = g * 16 + lax.iota(jnp.int32, 16)
        for k in range(K_C):
            plsc.store_scatter(qzb, [rows, jnp.full((16,), k, jnp.int32)],
                               es[k] * r)
        vqb[pl.ds(g * 16, 16)] = bb[pl.ds(g * 16, 16)] + m + _log_f32(z)
        return carry

    lax.fori_loop(0, ROWS // 16, group, 0)

    g0 = pl.multiple_of(c_lo + r0l, 8)
    pltpu.sync_copy(qzb, qz_out.at[pl.ds(g0, ROWS), :])
    pltpu.sync_copy(vqb, vq_out.at[pl.ds(g0, ROWS)])


_vq_kernel = functools.partial(
    pl.kernel,
    mesh=plsc.VectorSubcoreMesh(core_axis_name="c", subcore_axis_name="s"),
    compiler_params=pltpu.CompilerParams(needs_layout_passes=False),
    out_type=[
        jax.ShapeDtypeStruct((I_T, K_C), jnp.float32),
        jax.ShapeDtypeStruct((I_T,), jnp.float32),
    ],
    scratch_types=[
        pltpu.VMEM((LBL_W,), jnp.int32),
        pltpu.VMEM((LBL_W,), jnp.int32),
        pltpu.VMEM((LBL_W,), jnp.int32),
        pltpu.VMEM((J_W,), jnp.float32),
        pltpu.VMEM((J_W,), jnp.float32),
        pltpu.VMEM((64,), jnp.float32),
        pltpu.VMEM((NCH, CH), jnp.int32),
        pltpu.VMEM((NCH, CH), jnp.float32),
        pltpu.VMEM((NCH, CH), jnp.int32),
        pltpu.VMEM((NCH, CH), jnp.float32),
        pltpu.VMEM((K_C * ROWS,), jnp.float32),
        pltpu.VMEM((ROWS, K_C), jnp.float32),
        pltpu.VMEM((ROWS,), jnp.float32),
        pltpu.VMEM((ROWS,), jnp.float32),
        pltpu.VMEM_SHARED((K_C * IC_PAD,), jnp.float32),
        pltpu.VMEM_SHARED((IC_PAD,), jnp.float32),
        pltpu.VMEM_SHARED((1024,), jnp.float32),
        pltpu.VMEM_SHARED((1024,), jnp.float32),
        pltpu.SemaphoreType.DMA,
        pltpu.SemaphoreType.DMA,
    ],
)(_vq_body)


def kernel(ii, jj, y, snr_logit):
    ii = ii.astype(jnp.int32)
    jj = jj.astype(jnp.int32)
    y = y.astype(jnp.int32)
    zer = jnp.zeros((ZW,), jnp.float32)
    qz, vq = _vq_kernel(ii, jj, y, snr_logit, zer)
    return qz, vq


# speedup vs baseline: 1.0275x; 1.0275x over previous
import functools

import jax
import jax.numpy as jnp
from jax import lax
from jax.experimental import pallas as pl
from jax.experimental.pallas import tpu as pltpu
from jax.experimental.pallas import tpu_sc as plsc

I_T = 10000
J_W = 1000
K_C = 32
N_L = 10000

NC = 2
NS = 16

HALF = I_T // NC
IC_PAD = 5120
LBL_OWN = N_L // NS
LBL_W = 640
CH = 64
NCH = LBL_W // CH
ROWS = 320
ZW = K_C * IC_PAD // NS
LN2 = 0.6931471805599453


def _ad_body(s_ref, a_ref, d_ref):
    s = s_ref[...]
    sig = jax.nn.sigmoid(s)
    noi = jax.nn.sigmoid(-s)
    tot = sig + noi
    a = jnp.log(noi / (K_C * tot))
    b = jnp.log((sig + noi / K_C) / tot)
    a_ref[...] = a
    d_ref[...] = b - a


def _log_f32(x):
    bits = plsc.bitcast(x, jnp.int32)
    e = (bits >> 23) - 127
    m = plsc.bitcast((bits & 0x7FFFFF) | 0x3F800000, jnp.float32)
    s = (m - 1.0) / (m + 1.0)
    s2 = s * s
    p = 1.0 + s2 * (0.3333333333 + s2 * (0.2 + s2 * (0.14285714 + s2 * 0.11111111)))
    return e.astype(jnp.float32) * LN2 + 2.0 * s * p


def _vq_body(ii_hbm, jj_hbm, y_hbm, a_hbm, d_hbm, zer_hbm,
             qz_out, vq_out,
             ii_v, jj_v, y_v, a_v, d_v, idx_b, val_b, iib_b, av_b,
             colb, qzb, bb, vqb,
             scat_sh, base_sh, sem, ssem):
    c = lax.axis_index("c")
    s = lax.axis_index("s")
    own_lo = s * LBL_OWN
    l0 = jnp.minimum(own_lo & ~7, N_L - LBL_W)
    l0 = pl.multiple_of(l0, 8)
    c_lo = c * HALF

    cps = [
        pltpu.async_copy(ii_hbm.at[pl.ds(l0, LBL_W)], ii_v, sem),
        pltpu.async_copy(jj_hbm.at[pl.ds(l0, LBL_W)], jj_v, sem),
        pltpu.async_copy(y_hbm.at[pl.ds(l0, LBL_W)], y_v, sem),
        pltpu.async_copy(a_hbm, a_v, sem),
        pltpu.async_copy(d_hbm, d_v, sem),
        pltpu.async_copy(zer_hbm, scat_sh.at[pl.ds(s * ZW, ZW)], sem),
        pltpu.async_copy(zer_hbm.at[pl.ds(0, 640)],
                         base_sh.at[pl.ds((s % 8) * 640, 640)], sem),
    ]
    for cp in cps:
        cp.wait()
    plsc.subcore_barrier()

    zero = jnp.zeros((16,), jnp.float32)
    scatter_cps = []
    for chunk in range(NCH):
        for v in range(CH // 16):
            off = chunk * CH + v * 16
            iiv = ii_v[pl.ds(off, 16)]
            jjv = jj_v[pl.ds(off, 16)]
            yv = y_v[pl.ds(off, 16)]
            av = plsc.load_gather(a_v, [jjv])
            dv = plsc.load_gather(d_v, [jjv])
            g = l0 + off + lax.iota(jnp.int32, 16)
            col = iiv - c_lo
            ok = (g >= own_lo) & (g < own_lo + LBL_OWN) \
                & (col >= 0) & (col < HALF)
            colc = jnp.where(ok, col, 0)
            neg1 = jnp.full((16,), -1, jnp.int32)
            idx_b[chunk, pl.ds(v * 16, 16)] = jnp.where(
                ok, yv * IC_PAD + colc, neg1)
            val_b[chunk, pl.ds(v * 16, 16)] = jnp.where(ok, dv, zero)
            iib_b[chunk, pl.ds(v * 16, 16)] = jnp.where(ok, colc, neg1)
            av_b[chunk, pl.ds(v * 16, 16)] = jnp.where(ok, av, zero)
        scatter_cps.append(
            pltpu.async_copy(
                val_b.at[chunk],
                scat_sh.at[plsc.Indices(idx_b.at[chunk], ignored_value=-1)],
                ssem, add=True))
        scatter_cps.append(
            pltpu.async_copy(
                av_b.at[chunk],
                base_sh.at[plsc.Indices(iib_b.at[chunk], ignored_value=-1)],
                ssem, add=True))
    for cp in scatter_cps:
        cp.wait()
    plsc.subcore_barrier()

    r0l = jnp.minimum(s * 313, HALF - ROWS) & ~7
    r0l = pl.multiple_of(r0l, 8)
    col_cps = [
        pltpu.async_copy(scat_sh.at[pl.ds(k * IC_PAD + r0l, ROWS)],
                         colb.at[pl.ds(k * ROWS, ROWS)], sem)
        for k in range(K_C)
    ]
    col_cps.append(pltpu.async_copy(base_sh.at[pl.ds(r0l, ROWS)], bb, sem))
    for cp in col_cps:
        cp.wait()

    def _tree(xs, op):
        while len(xs) > 1:
            xs = [op(xs[i], xs[i + 1]) for i in range(0, len(xs) - 1, 2)] + (
                [xs[-1]] if len(xs) % 2 else [])
        return xs[0]

    def group(g, carry):
        vs = [colb[pl.ds(k * ROWS + g * 16, 16)] for k in range(K_C)]
        m = _tree(vs, jnp.maximum)
        es = [jnp.exp(v - m) for v in vs]
        z = _tree(es, lambda a, b: a + b)
        r = 1.0 / z
        rows = g * 16 + lax.iota(jnp.int32, 16)
        for k in range(K_C):
            plsc.store_scatter(qzb, [rows, jnp.full((16,), k, jnp.int32)],
                               es[k] * r)
        vqb[pl.ds(g * 16, 16)] = bb[pl.ds(g * 16, 16)] + m + _log_f32(z)
        return carry

    lax.fori_loop(0, ROWS // 16, group, 0)

    g0 = pl.multiple_of(c_lo + r0l, 8)
    pltpu.sync_copy(qzb, qz_out.at[pl.ds(g0, ROWS), :])
    pltpu.sync_copy(vqb, vq_out.at[pl.ds(g0, ROWS)])


_vq_kernel = functools.partial(
    pl.kernel,
    mesh=plsc.VectorSubcoreMesh(core_axis_name="c", subcore_axis_name="s"),
    compiler_params=pltpu.CompilerParams(needs_layout_passes=False),
    out_type=[
        jax.ShapeDtypeStruct((I_T, K_C), jnp.float32),
        jax.ShapeDtypeStruct((I_T,), jnp.float32),
    ],
    scratch_types=[
        pltpu.VMEM((LBL_W,), jnp.int32),
        pltpu.VMEM((LBL_W,), jnp.int32),
        pltpu.VMEM((LBL_W,), jnp.int32),
        pltpu.VMEM((J_W,), jnp.float32),
        pltpu.VMEM((J_W,), jnp.float32),
        pltpu.VMEM((NCH, CH), jnp.int32),
        pltpu.VMEM((NCH, CH), jnp.float32),
        pltpu.VMEM((NCH, CH), jnp.int32),
        pltpu.VMEM((NCH, CH), jnp.float32),
        pltpu.VMEM((K_C * ROWS,), jnp.float32),
        pltpu.VMEM((ROWS, K_C), jnp.float32),
        pltpu.VMEM((ROWS,), jnp.float32),
        pltpu.VMEM((ROWS,), jnp.float32),
        pltpu.VMEM_SHARED((K_C * IC_PAD,), jnp.float32),
        pltpu.VMEM_SHARED((IC_PAD,), jnp.float32),
        pltpu.SemaphoreType.DMA,
        pltpu.SemaphoreType.DMA,
    ],
)(_vq_body)


def kernel(ii, jj, y, snr_logit):
    ii = ii.astype(jnp.int32)
    jj = jj.astype(jnp.int32)
    y = y.astype(jnp.int32)

    a_p, d_p = pl.pallas_call(
        _ad_body,
        out_shape=[jax.ShapeDtypeStruct((J_W,), jnp.float32)] * 2,
    )(snr_logit)

    zer = jnp.zeros((ZW,), jnp.float32)
    qz, vq = _vq_kernel(ii, jj, y, a_p, d_p, zer)
    return qz, vq
